# Initial kernel scaffold; baseline (speedup 1.0000x reference)
import jax
import jax.numpy as jnp
from jax.experimental import pallas as pl


def kernel(x, edge_index, batch, W1, b1, g1, be1, W2, b2, g2, be2, W3, b3, g3, be3, centroids):
    raise NotImplementedError("write your pallas kernel here")


# speedup vs baseline: 28.8749x; 28.8749x over previous
import functools

import jax
import jax.numpy as jnp
from jax import lax
from jax.experimental import pallas as pl
from jax.experimental.pallas import tpu as pltpu
from jax.experimental.pallas import tpu_sc as plsc

N = 10000
E = 320000
D = 128
H = 64
G = 64
NTILES = 32
EPT = E // NTILES
CHUNK = 80
NCHUNK = EPT // CHUNK
RPS = N // 16

_mesh = plsc.VectorSubcoreMesh(core_axis_name="c", subcore_axis_name="s")
_sc_params = pltpu.CompilerParams(needs_layout_passes=False,
                                  use_tc_tiling_on_sc=False)


@functools.partial(
    pl.kernel,
    out_type=jax.ShapeDtypeStruct((NTILES, N), jnp.float32),
    scratch_types=[
        pltpu.VMEM((EPT,), jnp.int32),
        pltpu.VMEM((N,), jnp.float32),
    ],
    mesh=_mesh,
    compiler_params=_sc_params,
)
def _sc_degree(dst_hbm, zeros_hbm, out_hbm, idx_v, hist_v):
    wid = lax.axis_index("s") * 2 + lax.axis_index("c")
    pltpu.sync_copy(dst_hbm.at[wid], idx_v)
    pltpu.sync_copy(zeros_hbm, hist_v)

    def body(i, carry):
        idx = idx_v[pl.ds(i * 16, 16)]
        plsc.addupdate_scatter(hist_v, [idx], jnp.ones((16,), jnp.float32))
        return carry

    lax.fori_loop(0, EPT // 16, body, 0)
    pltpu.sync_copy(hist_v, out_hbm.at[wid])


@functools.partial(
    pl.kernel,
    out_type=jax.ShapeDtypeStruct((2, N, H), jnp.float32),
    scratch_types=[
        pltpu.VMEM((NCHUNK, CHUNK), jnp.int32),
        pltpu.VMEM((NCHUNK, CHUNK), jnp.int32),
        pltpu.VMEM((CHUNK, H), jnp.float32),
        pltpu.VMEM((CHUNK, H), jnp.float32),
        pltpu.VMEM_SHARED((N, H), jnp.float32),
        pltpu.SemaphoreType.DMA,
        pltpu.SemaphoreType.DMA,
    ],
    mesh=_mesh,
    compiler_params=_sc_params,
)
def _sc_agg(h_hbm, src_hbm, dst_hbm, zeros_hbm, out_hbm,
            src_v, dst_v, m0, m1, acc, sem0, sem1):
    c = lax.axis_index("c")
    s = lax.axis_index("s")
    wid = s * 2 + c
    pltpu.sync_copy(src_hbm.at[wid], src_v)
    pltpu.sync_copy(dst_hbm.at[wid], dst_v)
    pltpu.sync_copy(zeros_hbm, acc.at[pl.ds(s * RPS, RPS)])
    plsc.subcore_barrier()

    pltpu.async_copy(h_hbm.at[src_v.at[0]], m0, sem0)

    def body(j, carry):
        @pl.when(j % 2 == 0)
        def _even():
            @pl.when(j + 1 < NCHUNK)
            def _():
                pltpu.async_copy(h_hbm.at[src_v.at[j + 1]], m1, sem1)
            pltpu.make_async_copy(h_hbm.at[src_v.at[j]], m0, sem0).wait()
            pltpu.sync_copy(m0, acc.at[dst_v.at[j]], add=True)

        @pl.when(j % 2 == 1)
        def _odd():
            @pl.when(j + 1 < NCHUNK)
            def _():
                pltpu.async_copy(h_hbm.at[src_v.at[j + 1]], m0, sem0)
            pltpu.make_async_copy(h_hbm.at[src_v.at[j]], m1, sem1).wait()
            pltpu.sync_copy(m1, acc.at[dst_v.at[j]], add=True)

        return carry

    lax.fori_loop(0, NCHUNK, body, 0)
    plsc.subcore_barrier()
    pltpu.sync_copy(acc.at[pl.ds(s * RPS, RPS)],
                    out_hbm.at[c, pl.ds(s * RPS, RPS)])


def _tc_prep_body(hists_ref, x_ref, w_ref, h_ref, dis_ref):
    deg = jnp.sum(hists_ref[...], axis=0) + 1.0
    dis = lax.rsqrt(deg)[:, None]
    dis_ref[...] = dis
    h_ref[...] = dis * jnp.dot(x_ref[...], w_ref[...],
                               preferred_element_type=jnp.float32)


def _tc_mid_body(p0_ref, p1_ref, hp_ref, dis_ref, g_ref, be_ref, w_ref, out_ref):
    dis = dis_ref[...]
    a = dis * (p0_ref[...] + p1_ref[...] + hp_ref[...])
    m = jnp.mean(a, axis=0, keepdims=True)
    d = a - m
    v = jnp.mean(d * d, axis=0, keepdims=True)
    z = d * lax.rsqrt(v + 1e-5) * g_ref[...][None, :] + be_ref[...][None, :]
    z = jnp.maximum(z, 0.0)
    out_ref[...] = dis * jnp.dot(z, w_ref[...], preferred_element_type=jnp.float32)


def _tc_final_body(p0_ref, p1_ref, hp_ref, dis_ref, g_ref, be_ref,
                   batch_ref, ck_ref, out_ref):
    dis = dis_ref[...]
    a = dis * (p0_ref[...] + p1_ref[...] + hp_ref[...])
    m = jnp.mean(a, axis=0, keepdims=True)
    d = a - m
    v = jnp.mean(d * d, axis=0, keepdims=True)
    z = d * lax.rsqrt(v + 1e-5) * g_ref[...][None, :] + be_ref[...][None, :]
    gid = lax.broadcasted_iota(jnp.int32, (N, G), 1)
    oh = (batch_ref[...] == gid).astype(jnp.float32)
    sums = lax.dot_general(oh, z, (((0,), (0,)), ((), ())),
                           preferred_element_type=jnp.float32)
    cnt = jnp.sum(oh, axis=0)
    emb = sums / jnp.maximum(cnt, 1.0)[:, None]
    e2 = jnp.sum(emb * emb, axis=1, keepdims=True)
    best = None
    for k in range(3):
        ck = ck_ref[k]
        cross = lax.dot_general(emb, ck, (((1,), (1,)), ((), ())),
                                preferred_element_type=jnp.float32)
        d2 = e2 - 2.0 * cross + jnp.sum(ck * ck, axis=1)[None, :]
        best = d2 if best is None else jnp.minimum(best, d2)
    out_ref[...] = -best


_tc_prep = pl.pallas_call(
    _tc_prep_body,
    out_shape=[jax.ShapeDtypeStruct((N, H), jnp.float32),
               jax.ShapeDtypeStruct((N, 1), jnp.float32)],
)

_tc_mid = pl.pallas_call(
    _tc_mid_body,
    out_shape=jax.ShapeDtypeStruct((N, H), jnp.float32),
)

_tc_final = pl.pallas_call(
    _tc_final_body,
    out_shape=jax.ShapeDtypeStruct((G, 10), jnp.float32),
)


def kernel(x, edge_index, batch, W1, b1, g1, be1, W2, b2, g2, be2,
           W3, b3, g3, be3, centroids):
    src = edge_index[0].astype(jnp.int32)
    dst = edge_index[1].astype(jnp.int32)
    src3 = src.reshape(NTILES, NCHUNK, CHUNK)
    dst3 = dst.reshape(NTILES, NCHUNK, CHUNK)
    dst2 = dst.reshape(NTILES, EPT)
    batch2 = batch.astype(jnp.int32).reshape(N, 1)
    ck = centroids.reshape(10, 3, H).transpose(1, 0, 2)
    zeros_n = jnp.zeros((N,), jnp.float32)
    zeros_rh = jnp.zeros((RPS, H), jnp.float32)

    hists = _sc_degree(dst2, zeros_n)
    h1p, dis = _tc_prep(hists, x, W1)
    p = _sc_agg(h1p, src3, dst3, zeros_rh)
    h2p = _tc_mid(p[0], p[1], h1p, dis, g1, be1, W2)
    p = _sc_agg(h2p, src3, dst3, zeros_rh)
    h3p = _tc_mid(p[0], p[1], h2p, dis, g2, be2, W3)
    p = _sc_agg(h3p, src3, dst3, zeros_rh)
    logits = _tc_final(p[0], p[1], h3p, dis, g3, be3, batch2, ck)
    return logits


# speedup vs baseline: 36.5269x; 1.2650x over previous
import functools

import jax
import jax.numpy as jnp
from jax import lax
from jax.experimental import pallas as pl
from jax.experimental.pallas import tpu as pltpu
from jax.experimental.pallas import tpu_sc as plsc

N = 10000
E = 320000
D = 128
H = 64
G = 64
NTILES = 32
EPT = E // NTILES
CHUNK = 80
NCHUNK = EPT // CHUNK
RPS = N // 16
NBUF = 5

_mesh = plsc.VectorSubcoreMesh(core_axis_name="c", subcore_axis_name="s")
_sc_params = pltpu.CompilerParams(needs_layout_passes=False,
                                  use_tc_tiling_on_sc=False)


@functools.partial(
    pl.kernel,
    out_type=jax.ShapeDtypeStruct((NTILES, N), jnp.float32),
    scratch_types=[
        pltpu.VMEM((EPT,), jnp.int32),
        pltpu.VMEM((N,), jnp.float32),
    ],
    mesh=_mesh,
    compiler_params=_sc_params,
)
def _sc_degree(dst_hbm, zeros_hbm, out_hbm, idx_v, hist_v):
    wid = lax.axis_index("s") * 2 + lax.axis_index("c")
    pltpu.sync_copy(dst_hbm.at[wid], idx_v)
    pltpu.sync_copy(zeros_hbm, hist_v)

    def body(i, carry):
        idx = idx_v[pl.ds(i * 16, 16)]
        plsc.addupdate_scatter(hist_v, [idx], jnp.ones((16,), jnp.float32))
        return carry

    lax.fori_loop(0, EPT // 16, body, 0)
    pltpu.sync_copy(hist_v, out_hbm.at[wid])


@functools.partial(
    pl.kernel,
    out_type=jax.ShapeDtypeStruct((2, N, H), jnp.float32),
    scratch_types=[
        pltpu.VMEM((NCHUNK, CHUNK), jnp.int32),
        pltpu.VMEM((NCHUNK, CHUNK), jnp.int32),
        pltpu.VMEM((NBUF, CHUNK, H), jnp.float32),
        pltpu.VMEM_SHARED((N, H), jnp.float32),
        pltpu.SemaphoreType.DMA((NBUF,)),
        pltpu.SemaphoreType.DMA((NBUF,)),
    ],
    mesh=_mesh,
    compiler_params=_sc_params,
)
def _sc_agg(h_hbm, src_hbm, dst_hbm, zeros_hbm, out_hbm,
            src_v, dst_v, msg, acc, semg, sems):
    c = lax.axis_index("c")
    s = lax.axis_index("s")
    wid = s * 2 + c
    pltpu.sync_copy(src_hbm.at[wid], src_v)
    pltpu.sync_copy(dst_hbm.at[wid], dst_v)
    pltpu.sync_copy(zeros_hbm, acc.at[pl.ds(s * RPS, RPS)])
    plsc.subcore_barrier()

    for b in range(3):
        pltpu.async_copy(h_hbm.at[src_v.at[b]], msg.at[b], semg.at[b])

    @pl.loop(0, NCHUNK, step=NBUF)
    def _outer(j0):
        for b in range(NBUF):
            j = j0 + b
            bn = (b + 3) % NBUF

            @pl.when(j + 3 < NCHUNK)
            def _issue():
                @pl.when(j >= 2)
                def _drain_prev():
                    pltpu.make_async_copy(
                        msg.at[bn], acc.at[dst_v.at[j]], sems.at[bn]).wait()
                pltpu.async_copy(h_hbm.at[src_v.at[j + 3]], msg.at[bn],
                                 semg.at[bn])

            pltpu.make_async_copy(h_hbm.at[src_v.at[j]], msg.at[b],
                                  semg.at[b]).wait()
            pltpu.async_copy(msg.at[b], acc.at[dst_v.at[j]], sems.at[b],
                             add=True)

    for b in range(NBUF):
        pltpu.make_async_copy(msg.at[b], acc.at[dst_v.at[0]], sems.at[b]).wait()
    plsc.subcore_barrier()
    pltpu.sync_copy(acc.at[pl.ds(s * RPS, RPS)],
                    out_hbm.at[c, pl.ds(s * RPS, RPS)])


def _tc_prep_body(hists_ref, x_ref, w_ref, h_ref, dis_ref):
    deg = jnp.sum(hists_ref[...], axis=0) + 1.0
    dis = lax.rsqrt(deg)[:, None]
    dis_ref[...] = dis
    h_ref[...] = dis * jnp.dot(x_ref[...], w_ref[...],
                               preferred_element_type=jnp.float32)


def _tc_mid_body(p0_ref, p1_ref, hp_ref, dis_ref, g_ref, be_ref, w_ref, out_ref):
    dis = dis_ref[...]
    a = dis * (p0_ref[...] + p1_ref[...] + hp_ref[...])
    m = jnp.mean(a, axis=0, keepdims=True)
    d = a - m
    v = jnp.mean(d * d, axis=0, keepdims=True)
    z = d * lax.rsqrt(v + 1e-5) * g_ref[...][None, :] + be_ref[...][None, :]
    z = jnp.maximum(z, 0.0)
    out_ref[...] = dis * jnp.dot(z, w_ref[...], preferred_element_type=jnp.float32)


def _tc_final_body(p0_ref, p1_ref, hp_ref, dis_ref, g_ref, be_ref,
                   batch_ref, ck_ref, out_ref):
    dis = dis_ref[...]
    a = dis * (p0_ref[...] + p1_ref[...] + hp_ref[...])
    m = jnp.mean(a, axis=0, keepdims=True)
    d = a - m
    v = jnp.mean(d * d, axis=0, keepdims=True)
    z = d * lax.rsqrt(v + 1e-5) * g_ref[...][None, :] + be_ref[...][None, :]
    gid = lax.broadcasted_iota(jnp.int32, (N, G), 1)
    oh = (batch_ref[...] == gid).astype(jnp.float32)
    sums = lax.dot_general(oh, z, (((0,), (0,)), ((), ())),
                           preferred_element_type=jnp.float32)
    cnt = jnp.sum(oh, axis=0)
    emb = sums / jnp.maximum(cnt, 1.0)[:, None]
    e2 = jnp.sum(emb * emb, axis=1, keepdims=True)
    best = None
    for k in range(3):
        ck = ck_ref[k]
        cross = lax.dot_general(emb, ck, (((1,), (1,)), ((), ())),
                                preferred_element_type=jnp.float32)
        d2 = e2 - 2.0 * cross + jnp.sum(ck * ck, axis=1)[None, :]
        best = d2 if best is None else jnp.minimum(best, d2)
    out_ref[...] = -best


_tc_prep = pl.pallas_call(
    _tc_prep_body,
    out_shape=[jax.ShapeDtypeStruct((N, H), jnp.float32),
               jax.ShapeDtypeStruct((N, 1), jnp.float32)],
)

_tc_mid = pl.pallas_call(
    _tc_mid_body,
    out_shape=jax.ShapeDtypeStruct((N, H), jnp.float32),
)

_tc_final = pl.pallas_call(
    _tc_final_body,
    out_shape=jax.ShapeDtypeStruct((G, 10), jnp.float32),
)


def kernel(x, edge_index, batch, W1, b1, g1, be1, W2, b2, g2, be2,
           W3, b3, g3, be3, centroids):
    src = edge_index[0].astype(jnp.int32)
    dst = edge_index[1].astype(jnp.int32)
    src3 = src.reshape(NTILES, NCHUNK, CHUNK)
    dst3 = dst.reshape(NTILES, NCHUNK, CHUNK)
    dst2 = dst.reshape(NTILES, EPT)
    batch2 = batch.astype(jnp.int32).reshape(N, 1)
    ck = centroids.reshape(10, 3, H).transpose(1, 0, 2)
    zeros_n = jnp.zeros((N,), jnp.float32)
    zeros_rh = jnp.zeros((RPS, H), jnp.float32)

    hists = _sc_degree(dst2, zeros_n)
    h1p, dis = _tc_prep(hists, x, W1)
    p = _sc_agg(h1p, src3, dst3, zeros_rh)
    h2p = _tc_mid(p[0], p[1], h1p, dis, g1, be1, W2)
    p = _sc_agg(h2p, src3, dst3, zeros_rh)
    h3p = _tc_mid(p[0], p[1], h2p, dis, g2, be2, W3)
    p = _sc_agg(h3p, src3, dst3, zeros_rh)
    logits = _tc_final(p[0], p[1], h3p, dis, g3, be3, batch2, ck)
    return logits


# speedup vs baseline: 39.0651x; 1.0695x over previous
import functools

import jax
import jax.numpy as jnp
from jax import lax
from jax.experimental import pallas as pl
from jax.experimental.pallas import tpu as pltpu
from jax.experimental.pallas import tpu_sc as plsc

N = 10000
E = 320000
D = 128
H = 64
G = 64
NTILES = 32
EPT = E // NTILES
CHUNK = 80
NCHUNK = EPT // CHUNK
RPS = N // 16
NBUF = 5

_mesh = plsc.VectorSubcoreMesh(core_axis_name="c", subcore_axis_name="s")
_sc_params = pltpu.CompilerParams(needs_layout_passes=False,
                                  use_tc_tiling_on_sc=False)


@functools.partial(
    pl.kernel,
    out_type=jax.ShapeDtypeStruct((NTILES, N), jnp.float32),
    scratch_types=[
        pltpu.VMEM((EPT,), jnp.int32),
        pltpu.VMEM((N,), jnp.float32),
    ],
    mesh=_mesh,
    compiler_params=_sc_params,
)
def _sc_degree(dst_hbm, zeros_hbm, out_hbm, idx_v, hist_v):
    wid = lax.axis_index("s") * 2 + lax.axis_index("c")
    pltpu.sync_copy(dst_hbm.at[wid], idx_v)
    pltpu.sync_copy(zeros_hbm, hist_v)

    def body(i, carry):
        idx = idx_v[pl.ds(i * 16, 16)]
        plsc.addupdate_scatter(hist_v, [idx], jnp.ones((16,), jnp.float32))
        return carry

    lax.fori_loop(0, EPT // 16, body, 0)
    pltpu.sync_copy(hist_v, out_hbm.at[wid])


@functools.partial(
    pl.kernel,
    out_type=jax.ShapeDtypeStruct((2, N, H), jnp.float32),
    scratch_types=[
        pltpu.VMEM((NCHUNK, CHUNK), jnp.int32),
        pltpu.VMEM((NCHUNK, CHUNK), jnp.int32),
        pltpu.VMEM((NBUF, CHUNK, H), jnp.float32),
        pltpu.VMEM_SHARED((N, H), jnp.float32),
        pltpu.SemaphoreType.DMA((NBUF,)),
        pltpu.SemaphoreType.DMA((NBUF,)),
    ],
    mesh=_mesh,
    compiler_params=_sc_params,
)
def _sc_agg(h_hbm, src_hbm, dst_hbm, zeros_hbm, out_hbm,
            src_v, dst_v, msg, acc, semg, sems):
    c = lax.axis_index("c")
    s = lax.axis_index("s")
    wid = s * 2 + c
    pltpu.sync_copy(src_hbm.at[wid], src_v)
    pltpu.sync_copy(dst_hbm.at[wid], dst_v)
    pltpu.sync_copy(zeros_hbm, acc.at[pl.ds(s * RPS, RPS)])
    plsc.subcore_barrier()

    for b in range(3):
        pltpu.async_copy(h_hbm.at[src_v.at[b]], msg.at[b], semg.at[b])

    @pl.loop(0, NCHUNK, step=NBUF)
    def _outer(j0):
        for b in range(NBUF):
            j = j0 + b
            bn = (b + 3) % NBUF

            @pl.when(j + 3 < NCHUNK)
            def _issue():
                @pl.when(j >= 2)
                def _drain_prev():
                    pltpu.make_async_copy(
                        msg.at[bn], acc.at[dst_v.at[j]], sems.at[bn]).wait()
                pltpu.async_copy(h_hbm.at[src_v.at[j + 3]], msg.at[bn],
                                 semg.at[bn])

            pltpu.make_async_copy(h_hbm.at[src_v.at[j]], msg.at[b],
                                  semg.at[b]).wait()
            pltpu.async_copy(msg.at[b], acc.at[dst_v.at[j]], sems.at[b],
                             add=True)

    for b in range(NBUF):
        pltpu.make_async_copy(msg.at[b], acc.at[dst_v.at[0]], sems.at[b]).wait()
    plsc.subcore_barrier()
    pltpu.sync_copy(acc.at[pl.ds(s * RPS, RPS)],
                    out_hbm.at[c, pl.ds(s * RPS, RPS)])


def _tc_prep_body(hists_ref, x_ref, w_ref, h_ref, dis_ref):
    deg = jnp.sum(hists_ref[...], axis=0) + 1.0
    dis = lax.rsqrt(deg)[:, None]
    dis_ref[...] = dis
    h_ref[...] = dis * jnp.dot(x_ref[...], w_ref[...],
                               preferred_element_type=jnp.float32)


def _tc_mid_body(p_ref, hp_ref, dis_ref, g_ref, be_ref, w_ref, out_ref):
    dis = dis_ref[...]
    a = dis * (p_ref[0] + p_ref[1] + hp_ref[...])
    m = jnp.mean(a, axis=0, keepdims=True)
    d = a - m
    v = jnp.mean(d * d, axis=0, keepdims=True)
    z = d * lax.rsqrt(v + 1e-5) * g_ref[...][None, :] + be_ref[...][None, :]
    z = jnp.maximum(z, 0.0)
    out_ref[...] = dis * jnp.dot(z, w_ref[...], preferred_element_type=jnp.float32)


def _tc_final_body(p_ref, hp_ref, dis_ref, g_ref, be_ref,
                   batch_ref, ck_ref, out_ref):
    dis = dis_ref[...]
    a = dis * (p_ref[0] + p_ref[1] + hp_ref[...])
    m = jnp.mean(a, axis=0, keepdims=True)
    d = a - m
    v = jnp.mean(d * d, axis=0, keepdims=True)
    z = d * lax.rsqrt(v + 1e-5) * g_ref[...][None, :] + be_ref[...][None, :]
    gid = lax.broadcasted_iota(jnp.int32, (N, G), 1)
    oh = (batch_ref[...] == gid).astype(jnp.float32)
    sums = lax.dot_general(oh, z, (((0,), (0,)), ((), ())),
                           preferred_element_type=jnp.float32)
    cnt = jnp.sum(oh, axis=0)
    emb = sums / jnp.maximum(cnt, 1.0)[:, None]
    e2 = jnp.sum(emb * emb, axis=1, keepdims=True)
    best = None
    for k in range(3):
        ck = ck_ref[k]
        cross = lax.dot_general(emb, ck, (((1,), (1,)), ((), ())),
                                preferred_element_type=jnp.float32)
        d2 = e2 - 2.0 * cross + jnp.sum(ck * ck, axis=1)[None, :]
        best = d2 if best is None else jnp.minimum(best, d2)
    out_ref[...] = -best


_tc_prep = pl.pallas_call(
    _tc_prep_body,
    out_shape=[jax.ShapeDtypeStruct((N, H), jnp.float32),
               jax.ShapeDtypeStruct((N, 1), jnp.float32)],
)

_tc_mid = pl.pallas_call(
    _tc_mid_body,
    out_shape=jax.ShapeDtypeStruct((N, H), jnp.float32),
)

_tc_final = pl.pallas_call(
    _tc_final_body,
    out_shape=jax.ShapeDtypeStruct((G, 10), jnp.float32),
)


def kernel(x, edge_index, batch, W1, b1, g1, be1, W2, b2, g2, be2,
           W3, b3, g3, be3, centroids):
    src = edge_index[0].astype(jnp.int32)
    dst = edge_index[1].astype(jnp.int32)
    src3 = src.reshape(NTILES, NCHUNK, CHUNK)
    dst3 = dst.reshape(NTILES, NCHUNK, CHUNK)
    dst2 = dst.reshape(NTILES, EPT)
    batch2 = batch.astype(jnp.int32).reshape(N, 1)
    ck = centroids.reshape(10, 3, H).transpose(1, 0, 2)
    zeros_n = jnp.zeros((N,), jnp.float32)
    zeros_rh = jnp.zeros((RPS, H), jnp.float32)

    hists = _sc_degree(dst2, zeros_n)
    h1p, dis = _tc_prep(hists, x, W1)
    p = _sc_agg(h1p, src3, dst3, zeros_rh)
    h2p = _tc_mid(p, h1p, dis, g1, be1, W2)
    p = _sc_agg(h2p, src3, dst3, zeros_rh)
    h3p = _tc_mid(p, h2p, dis, g2, be2, W3)
    p = _sc_agg(h3p, src3, dst3, zeros_rh)
    logits = _tc_final(p, h3p, dis, g3, be3, batch2, ck)
    return logits
